# Initial kernel scaffold; baseline (speedup 1.0000x reference)
import jax
import jax.numpy as jnp
from jax.experimental import pallas as pl


def kernel(Coordinate3D, Feature512D, JointType, edge_attr, params, edge_index):
    raise NotImplementedError("write your pallas kernel here")


# speedup vs baseline: 1.6077x; 1.6077x over previous
import functools

import jax
import jax.numpy as jnp
from jax import lax
from jax.experimental import pallas as pl
from jax.experimental.pallas import tpu as pltpu
from jax.experimental.pallas import tpu_sc as plsc

N = 10000
E = 160000

NW = 32
NPW = 320
NPAD = NW * NPW
EPW = 5120
EPAD = NW * EPW
GB = 512
NB = EPW // GB
IB = 128

_mesh = plsc.VectorSubcoreMesh(core_axis_name="c", subcore_axis_name="s")
_sc_params = pltpu.CompilerParams(use_tc_tiling_on_sc=False)


def _relu(x):
    return jnp.maximum(x, 0.0)


def _sc_gather_sum(U, V, idx_d, idx_s, H):
    @functools.partial(
        pl.kernel,
        mesh=_mesh,
        compiler_params=_sc_params,
        out_type=jax.ShapeDtypeStruct((EPAD, H), jnp.float32),
        scratch_types=[
            pltpu.VMEM((GB // IB, IB), jnp.int32),
            pltpu.VMEM((GB // IB, IB), jnp.int32),
            pltpu.VMEM((GB, H), jnp.float32),
            pltpu.VMEM((GB, H), jnp.float32),
            pltpu.SemaphoreType.DMA,
        ],
    )
    def k(u_hbm, v_hbm, d_hbm, s_hbm, o_hbm, di, si, bu, bv, sem):
        w = lax.axis_index("s") * 2 + lax.axis_index("c")

        @pl.loop(0, NB)
        def _(b):
            off = w * EPW + b * GB
            roff = w * (EPW // IB) + b * (GB // IB)
            pltpu.sync_copy(d_hbm.at[pl.ds(roff, GB // IB)], di)
            pltpu.sync_copy(s_hbm.at[pl.ds(roff, GB // IB)], si)
            cps = []
            for j in range(GB // IB):
                sl = pl.ds(j * IB, IB)
                cps.append(pltpu.async_copy(u_hbm.at[di.at[j]], bu.at[sl], sem))
                cps.append(pltpu.async_copy(v_hbm.at[si.at[j]], bv.at[sl], sem))
            for cp in cps:
                cp.wait()

            @pl.loop(0, GB)
            def _(r):
                for c in range(H // 16):
                    cs = pl.ds(c * 16, 16)
                    bu[r, cs] = bu[r, cs] + bv[r, cs]

            pltpu.sync_copy(bu, o_hbm.at[pl.ds(off, GB)])

    return k(U, V, idx_d, idx_s)


def _sc_gather_pair(X, idx_d, idx_s, H):
    @functools.partial(
        pl.kernel,
        mesh=_mesh,
        compiler_params=_sc_params,
        out_type=(jax.ShapeDtypeStruct((EPAD, H), jnp.float32),
                  jax.ShapeDtypeStruct((EPAD, H), jnp.float32)),
        scratch_types=[
            pltpu.VMEM((GB // IB, IB), jnp.int32),
            pltpu.VMEM((GB // IB, IB), jnp.int32),
            pltpu.VMEM((GB, H), jnp.float32),
            pltpu.VMEM((GB, H), jnp.float32),
            pltpu.SemaphoreType.DMA,
        ],
    )
    def k(x_hbm, d_hbm, s_hbm, od_hbm, os_hbm, di, si, bu, bv, sem):
        w = lax.axis_index("s") * 2 + lax.axis_index("c")

        @pl.loop(0, NB)
        def _(b):
            off = w * EPW + b * GB
            roff = w * (EPW // IB) + b * (GB // IB)
            pltpu.sync_copy(d_hbm.at[pl.ds(roff, GB // IB)], di)
            pltpu.sync_copy(s_hbm.at[pl.ds(roff, GB // IB)], si)
            cps = []
            for j in range(GB // IB):
                sl = pl.ds(j * IB, IB)
                cps.append(pltpu.async_copy(x_hbm.at[di.at[j]], bu.at[sl], sem))
                cps.append(pltpu.async_copy(x_hbm.at[si.at[j]], bv.at[sl], sem))
            for cp in cps:
                cp.wait()
            pltpu.sync_copy(bu, od_hbm.at[pl.ds(off, GB)])
            pltpu.sync_copy(bv, os_hbm.at[pl.ds(off, GB)])

    return k(X, idx_d, idx_s)


def _sc_gather_rows(T, idx, H):
    @functools.partial(
        pl.kernel,
        mesh=_mesh,
        compiler_params=_sc_params,
        out_type=jax.ShapeDtypeStruct((EPAD, H), jnp.float32),
        scratch_types=[
            pltpu.VMEM((GB // IB, IB), jnp.int32),
            pltpu.VMEM((GB, H), jnp.float32),
            pltpu.SemaphoreType.DMA,
        ],
    )
    def k(t_hbm, i_hbm, o_hbm, di, bu, sem):
        w = lax.axis_index("s") * 2 + lax.axis_index("c")

        @pl.loop(0, NB)
        def _(b):
            off = w * EPW + b * GB
            roff = w * (EPW // IB) + b * (GB // IB)
            pltpu.sync_copy(i_hbm.at[pl.ds(roff, GB // IB)], di)
            cps = []
            for j in range(GB // IB):
                sl = pl.ds(j * IB, IB)
                cps.append(pltpu.async_copy(t_hbm.at[di.at[j]], bu.at[sl], sem))
            for cp in cps:
                cp.wait()
            pltpu.sync_copy(bu, o_hbm.at[pl.ds(off, GB)])

    return k(T, idx)


def _sc_segmax(M, dstp, bounds, H):
    @functools.partial(
        pl.kernel,
        mesh=_mesh,
        compiler_params=_sc_params,
        out_type=jax.ShapeDtypeStruct((NPAD, H), jnp.float32),
        scratch_types=[
            pltpu.VMEM((NW, 16), jnp.int32),
            pltpu.VMEM((GB,), jnp.int32),
            pltpu.VMEM((GB, H), jnp.float32),
            pltpu.VMEM((NPW + 1, H), jnp.float32),
        ],
    )
    def k(m_hbm, d_hbm, b_hbm, o_hbm, bnd, dsm, mbuf, acc):
        w = lax.axis_index("s") * 2 + lax.axis_index("c")
        pltpu.sync_copy(b_hbm, bnd)

        @pl.loop(0, NPW + 1)
        def _(r):
            for c in range(H // 16):
                acc[r, pl.ds(c * 16, 16)] = jnp.zeros((16,), jnp.float32)

        bv = bnd[w, pl.ds(0, 16)]
        start = bv[0]
        end = bv[1]
        nbase = w * NPW

        abase = (start // 8) * 8

        @pl.loop(0, EPAD // GB)
        def _(i):
            base = pl.multiple_of(abase + i * GB, 8)

            @pl.when(base < end)
            def _():
                pltpu.sync_copy(m_hbm.at[pl.ds(base, GB)], mbuf)
                pltpu.sync_copy(d_hbm.at[pl.ds(base, GB)], dsm)

                @pl.loop(0, GB // 16)
                def _(g):
                    dvec = dsm[pl.ds(g * 16, 16)] - nbase
                    for lane in range(16):
                        kk = base + g * 16 + lane
                        ok = jnp.logical_and(kk >= start, kk < end)
                        loc = jnp.where(ok, dvec[lane], NPW)
                        r = g * 16 + lane
                        for c in range(H // 16):
                            cs = pl.ds(c * 16, 16)
                            acc[loc, cs] = jnp.maximum(acc[loc, cs],
                                                       mbuf[r, cs])
        pltpu.sync_copy(acc.at[pl.ds(0, NPW)], o_hbm.at[pl.ds(nbase, NPW)])

    return k(M, dstp, bounds)


def _tc_head(coord, feat, jt, p):
    (wh1, bh1, wh2, bh2, a0, a1, a2, v0, v1, v2, b1) = p

    def body(c_r, f_r, j_r, wh1_r, bh1_r, wh2_r, bh2_r, a0_r, a1_r, a2_r,
             v0_r, v1_r, v2_r, b1_r, u_o, v_o):
        x0 = _relu(jnp.dot(c_r[...], wh1_r[...],
                           preferred_element_type=jnp.float32) + bh1_r[...])
        x0 = _relu(jnp.dot(x0, wh2_r[...],
                           preferred_element_type=jnp.float32) + bh2_r[...])
        du = jnp.dot(x0, a0_r[...], preferred_element_type=jnp.float32)
        du += jnp.dot(f_r[...], a1_r[...], preferred_element_type=jnp.float32)
        du += jnp.dot(j_r[...], a2_r[...], preferred_element_type=jnp.float32)
        dv = jnp.dot(x0, v0_r[...], preferred_element_type=jnp.float32)
        dv += jnp.dot(f_r[...], v1_r[...], preferred_element_type=jnp.float32)
        dv += jnp.dot(j_r[...], v2_r[...], preferred_element_type=jnp.float32)
        u_o[...] = du - dv + b1_r[...]
        v_o[...] = dv

    H = a0.shape[1]
    RB = 1000
    full = lambda a: pl.BlockSpec(a.shape, lambda i: tuple(0 for _ in a.shape))
    return pl.pallas_call(
        body,
        grid=(N // RB,),
        in_specs=[
            pl.BlockSpec((RB, 3), lambda i: (i, 0)),
            pl.BlockSpec((RB, 512), lambda i: (i, 0)),
            pl.BlockSpec((RB, 16), lambda i: (i, 0)),
            full(wh1), full(bh1), full(wh2), full(bh2),
            full(a0), full(a1), full(a2),
            full(v0), full(v1), full(v2), full(b1),
        ],
        out_specs=(pl.BlockSpec((RB, H), lambda i: (i, 0)),
                   pl.BlockSpec((RB, H), lambda i: (i, 0))),
        out_shape=(jax.ShapeDtypeStruct((N, H), jnp.float32),
                   jax.ShapeDtypeStruct((N, H), jnp.float32)),
    )(coord, feat, jt, wh1, bh1, wh2, bh2, a0, a1, a2, v0, v1, v2, b1)


def _tc_edge_proj(ea_s, we1, we2):
    def body(e_r, w1_r, w2_r, c1_o, c2_o):
        c1_o[...] = jnp.dot(e_r[...], w1_r[...],
                            preferred_element_type=jnp.float32)
        c2_o[...] = jnp.dot(e_r[...], w2_r[...],
                            preferred_element_type=jnp.float32)

    H = we1.shape[1]
    g = EPAD // GB
    return pl.pallas_call(
        body,
        grid=(g,),
        in_specs=[
            pl.BlockSpec((GB, 16), lambda i: (i, 0)),
            pl.BlockSpec((16, H), lambda i: (0, 0)),
            pl.BlockSpec((16, H), lambda i: (0, 0)),
        ],
        out_specs=(pl.BlockSpec((GB, H), lambda i: (i, 0)),
                   pl.BlockSpec((GB, H), lambda i: (i, 0))),
        out_shape=(jax.ShapeDtypeStruct((EPAD, H), jnp.float32),
                   jax.ShapeDtypeStruct((EPAD, H), jnp.float32)),
    )(ea_s, we1, we2)


def _tc_mlp2(G0, C, w2, b2):
    H = w2.shape[1]
    g = EPAD // GB

    if C is not None:
        def body(g_r, c_r, w_r, b_r, m_o):
            h = _relu(g_r[...] + c_r[...])
            m_o[...] = _relu(jnp.dot(h, w_r[...],
                                     preferred_element_type=jnp.float32)
                             + b_r[...])
        in_specs = [
            pl.BlockSpec((GB, H), lambda i: (i, 0)),
            pl.BlockSpec((GB, H), lambda i: (i, 0)),
            pl.BlockSpec((H, H), lambda i: (0, 0)),
            pl.BlockSpec((1, H), lambda i: (0, 0)),
        ]
        args = (G0, C, w2, b2)
    else:
        def body(g_r, w_r, b_r, m_o):
            h = _relu(g_r[...])
            m_o[...] = _relu(jnp.dot(h, w_r[...],
                                     preferred_element_type=jnp.float32)
                             + b_r[...])
        in_specs = [
            pl.BlockSpec((GB, H), lambda i: (i, 0)),
            pl.BlockSpec((H, H), lambda i: (0, 0)),
            pl.BlockSpec((1, H), lambda i: (0, 0)),
        ]
        args = (G0, w2, b2)

    return pl.pallas_call(
        body,
        grid=(g,),
        in_specs=in_specs,
        out_specs=pl.BlockSpec((GB, H), lambda i: (i, 0)),
        out_shape=jax.ShapeDtypeStruct((EPAD + GB, H), jnp.float32),
    )(*args)


def _tc_mlp_ref(Gd, Gs, C, wi, wj, b1, w2, b2):
    H = w2.shape[1]
    g = EPAD // GB

    if C is not None:
        def body(gd_r, gs_r, c_r, wi_r, wj_r, b1_r, w2_r, b2_r, m_o):
            xi = gd_r[...]
            dj = gs_r[...] - xi
            h = jnp.dot(xi, wi_r[...], preferred_element_type=jnp.float32)
            h += jnp.dot(dj, wj_r[...], preferred_element_type=jnp.float32)
            h = _relu(h + c_r[...] + b1_r[...])
            m_o[...] = _relu(jnp.dot(h, w2_r[...],
                                     preferred_element_type=jnp.float32)
                             + b2_r[...])
        in_specs = [
            pl.BlockSpec((GB, wi.shape[0]), lambda i: (i, 0)),
            pl.BlockSpec((GB, wi.shape[0]), lambda i: (i, 0)),
            pl.BlockSpec((GB, H), lambda i: (i, 0)),
            pl.BlockSpec(wi.shape, lambda i: (0, 0)),
            pl.BlockSpec(wj.shape, lambda i: (0, 0)),
            pl.BlockSpec((1, H), lambda i: (0, 0)),
            pl.BlockSpec((H, H), lambda i: (0, 0)),
            pl.BlockSpec((1, H), lambda i: (0, 0)),
        ]
        args = (Gd, Gs, C, wi, wj, b1, w2, b2)
    else:
        def body(gd_r, gs_r, wi_r, wj_r, b1_r, w2_r, b2_r, m_o):
            xi = gd_r[...]
            dj = gs_r[...] - xi
            h = jnp.dot(xi, wi_r[...], preferred_element_type=jnp.float32)
            h += jnp.dot(dj, wj_r[...], preferred_element_type=jnp.float32)
            h = _relu(h + b1_r[...])
            m_o[...] = _relu(jnp.dot(h, w2_r[...],
                                     preferred_element_type=jnp.float32)
                             + b2_r[...])
        in_specs = [
            pl.BlockSpec((GB, wi.shape[0]), lambda i: (i, 0)),
            pl.BlockSpec((GB, wi.shape[0]), lambda i: (i, 0)),
            pl.BlockSpec(wi.shape, lambda i: (0, 0)),
            pl.BlockSpec(wj.shape, lambda i: (0, 0)),
            pl.BlockSpec((1, H), lambda i: (0, 0)),
            pl.BlockSpec((H, H), lambda i: (0, 0)),
            pl.BlockSpec((1, H), lambda i: (0, 0)),
        ]
        args = (Gd, Gs, wi, wj, b1, w2, b2)

    return pl.pallas_call(
        body,
        grid=(g,),
        in_specs=in_specs,
        out_specs=pl.BlockSpec((GB, H), lambda i: (i, 0)),
        out_shape=jax.ShapeDtypeStruct((EPAD + GB, H), jnp.float32),
    )(*args)


def _tc_dense_relu(x, w, b):
    def body(x_r, w_r, b_r, o_r):
        o_r[...] = _relu(jnp.dot(x_r[...], w_r[...],
                                 preferred_element_type=jnp.float32) + b_r[...])

    return pl.pallas_call(
        body,
        out_shape=jax.ShapeDtypeStruct((N, w.shape[1]), jnp.float32),
    )(x, w, b)


def _tc_add(a, b):
    def body(a_r, b_r, o_r):
        o_r[...] = a_r[...] + b_r[...]

    return pl.pallas_call(
        body,
        out_shape=jax.ShapeDtypeStruct(a.shape, jnp.float32),
    )(a, b)


def _tc_tail(a5, a4, wo1, bo1, wo2, bo2):
    def body(a5_r, a4_r, w1_r, b1_r, w2_r, b2_r, y1_o, y2_o):
        xo = a5_r[...] + a4_r[...]
        y1 = _relu(jnp.dot(xo, w1_r[...],
                           preferred_element_type=jnp.float32) + b1_r[...])
        y1_o[...] = _relu(jnp.dot(y1, w1_r[...],
                                  preferred_element_type=jnp.float32) + b1_r[...])
        y2 = _relu(jnp.dot(xo, w2_r[...],
                           preferred_element_type=jnp.float32) + b2_r[...])
        y2_o[...] = _relu(jnp.dot(y2, w2_r[...],
                                  preferred_element_type=jnp.float32) + b2_r[...])

    H = wo1.shape[1]
    return pl.pallas_call(
        body,
        out_shape=(jax.ShapeDtypeStruct((N, H), jnp.float32),
                   jax.ShapeDtypeStruct((N, H), jnp.float32)),
    )(a5, a4, wo1, bo1, wo2, bo2)


def kernel(Coordinate3D, Feature512D, JointType, edge_attr, params, edge_index):
    i32 = jnp.int32
    src = edge_index[0].astype(i32)
    dst = edge_index[1].astype(i32)

    perm = jnp.argsort(dst)
    dst_s = dst[perm]
    src_s = src[perm]
    padE = EPAD - E
    zpad = jnp.zeros((padE,), i32)
    gd = jnp.concatenate([dst_s, zpad]).reshape(EPAD // IB, IB)
    gs = jnp.concatenate([src_s, zpad]).reshape(EPAD // IB, IB)
    gp = jnp.concatenate([perm.astype(i32), zpad]).reshape(EPAD // IB, IB)
    dstp = jnp.concatenate(
        [dst_s, jnp.full((padE + GB,), NPAD - 1, i32)])
    bnd1 = jnp.searchsorted(
        dstp[:EPAD], jnp.arange(0, NPW * (NW + 1), NPW)).astype(i32)
    bounds = jnp.zeros((NW, 16), i32)
    bounds = bounds.at[:, 0].set(bnd1[:NW]).at[:, 1].set(bnd1[1:])

    W1c = params["ece1_1"]["W"]
    au1 = W1c[:560]
    av1 = W1c[560:1120]
    we1 = W1c[1120:]
    b1_1 = params["ece1_1"]["b"].reshape(1, -1)
    we2 = params["ece2_1"]["W"][2 * 64:]
    r2 = lambda p: (p["W"], p["b"].reshape(1, -1))

    head_p = (params["h1"]["W"], params["h1"]["b"].reshape(1, -1),
              params["h2"]["W"], params["h2"]["b"].reshape(1, -1),
              au1[:32], au1[32:544], au1[544:],
              av1[:32], av1[32:544], av1[544:], b1_1)
    U1, V1 = _tc_head(Coordinate3D, Feature512D, JointType, head_p)

    ea_s = _sc_gather_rows(edge_attr, gp, 16)
    C1, C2 = _tc_edge_proj(ea_s, we1, we2)

    def conv1(U, V, C, p2, H):
        G0 = _sc_gather_sum(U, V, gd, gs, H)
        w2, b2 = r2(p2)
        M = _tc_mlp2(G0, C, w2, b2)
        return _sc_segmax(M, dstp, bounds, H)[:N]

    def conv_ref(x, p1, p2, C, cdim, H):
        W = p1["W"]
        wi = W[:cdim]
        wj = W[cdim:2 * cdim]
        b1 = p1["b"].reshape(1, -1)
        Gd, Gs = _sc_gather_pair(x, gd, gs, cdim)
        w2, b2 = r2(p2)
        M = _tc_mlp_ref(Gd, Gs, C, wi, wj, b1, w2, b2)
        return _sc_segmax(M, dstp, bounds, H)[:N]

    a1 = conv1(U1, V1, C1, params["ece1_2"], 64)
    a2 = conv_ref(a1, params["ece2_1"], params["ece2_2"], C2, 64, 64)
    x3 = _tc_dense_relu(a2, params["h3"]["W"],
                        params["h3"]["b"].reshape(1, -1))
    a3 = conv_ref(x3, params["ec1_1"], params["ec1_2"], None, 48, 48)
    a4 = conv_ref(a3, params["ec2_1"], params["ec2_2"], None, 48, 48)
    x5 = _tc_add(a4, a3)
    a5 = conv_ref(x5, params["ec3_1"], params["ec3_2"], None, 48, 48)

    wo1, bo1 = r2(params["out1"])
    wo2, bo2 = r2(params["out2"])
    return _tc_tail(a5, a4, wo1, bo1, wo2, bo2)


# speedup vs baseline: 1.6641x; 1.0350x over previous
import functools

import jax
import jax.numpy as jnp
from jax import lax
from jax.experimental import pallas as pl
from jax.experimental.pallas import tpu as pltpu
from jax.experimental.pallas import tpu_sc as plsc

N = 10000
E = 160000

NW = 32
NPW = 320
NPAD = NW * NPW
EPW = 5120
EPAD = NW * EPW
GB = 512
NB = EPW // GB
IB = 128

_mesh = plsc.VectorSubcoreMesh(core_axis_name="c", subcore_axis_name="s")
_sc_params = pltpu.CompilerParams(use_tc_tiling_on_sc=False)


def _relu(x):
    return jnp.maximum(x, 0.0)


GB2 = 256
NB2 = EPW // GB2
RPW = EPW // IB


def _sc_gather_sum(U, V, idx_d, idx_s, H):
    @functools.partial(
        pl.kernel,
        mesh=_mesh,
        compiler_params=_sc_params,
        out_type=jax.ShapeDtypeStruct((EPAD, H), jnp.float32),
        scratch_types=[
            pltpu.VMEM((RPW, IB), jnp.int32),
            pltpu.VMEM((RPW, IB), jnp.int32),
            pltpu.VMEM((GB2, H), jnp.float32),
            pltpu.VMEM((GB2, H), jnp.float32),
            pltpu.VMEM((GB2, H), jnp.float32),
            pltpu.VMEM((GB2, H), jnp.float32),
            pltpu.SemaphoreType.DMA,
            pltpu.SemaphoreType.DMA,
        ],
    )
    def k(u_hbm, v_hbm, d_hbm, s_hbm, o_hbm, dia, sia,
          bu0, bv0, bu1, bv1, sem0, sem1):
        w = lax.axis_index("s") * 2 + lax.axis_index("c")
        pltpu.sync_copy(d_hbm.at[pl.ds(w * RPW, RPW)], dia)
        pltpu.sync_copy(s_hbm.at[pl.ds(w * RPW, RPW)], sia)
        bufs = ((bu0, bv0, sem0), (bu1, bv1, sem1))

        def fire(b):
            bu, bv, sem = bufs[b % 2]
            cps = []
            for j in range(GB2 // IB):
                row = b * (GB2 // IB) + j
                sl = pl.ds(j * IB, IB)
                cps.append(pltpu.async_copy(u_hbm.at[dia.at[row]],
                                            bu.at[sl], sem))
                cps.append(pltpu.async_copy(v_hbm.at[sia.at[row]],
                                            bv.at[sl], sem))
            return cps

        pend = [fire(0), None]
        for b in range(NB2):
            if b + 1 < NB2:
                pend[(b + 1) % 2] = fire(b + 1)
            for cp in pend[b % 2]:
                cp.wait()
            bu, bv, _ = bufs[b % 2]

            @pl.loop(0, GB2)
            def _(r):
                for c in range(H // 16):
                    cs = pl.ds(c * 16, 16)
                    bu[r, cs] = bu[r, cs] + bv[r, cs]

            pltpu.sync_copy(bu, o_hbm.at[pl.ds(w * EPW + b * GB2, GB2)])

    return k(U, V, idx_d, idx_s)


def _sc_gather_pair(X, idx_d, idx_s, H):
    @functools.partial(
        pl.kernel,
        mesh=_mesh,
        compiler_params=_sc_params,
        out_type=(jax.ShapeDtypeStruct((EPAD, H), jnp.float32),
                  jax.ShapeDtypeStruct((EPAD, H), jnp.float32)),
        scratch_types=[
            pltpu.VMEM((RPW, IB), jnp.int32),
            pltpu.VMEM((RPW, IB), jnp.int32),
            pltpu.VMEM((GB2, H), jnp.float32),
            pltpu.VMEM((GB2, H), jnp.float32),
            pltpu.VMEM((GB2, H), jnp.float32),
            pltpu.VMEM((GB2, H), jnp.float32),
            pltpu.SemaphoreType.DMA,
            pltpu.SemaphoreType.DMA,
        ],
    )
    def k(x_hbm, d_hbm, s_hbm, od_hbm, os_hbm, dia, sia,
          bu0, bv0, bu1, bv1, sem0, sem1):
        w = lax.axis_index("s") * 2 + lax.axis_index("c")
        pltpu.sync_copy(d_hbm.at[pl.ds(w * RPW, RPW)], dia)
        pltpu.sync_copy(s_hbm.at[pl.ds(w * RPW, RPW)], sia)
        bufs = ((bu0, bv0, sem0), (bu1, bv1, sem1))

        def fire(b):
            bu, bv, sem = bufs[b % 2]
            cps = []
            for j in range(GB2 // IB):
                row = b * (GB2 // IB) + j
                sl = pl.ds(j * IB, IB)
                cps.append(pltpu.async_copy(x_hbm.at[dia.at[row]],
                                            bu.at[sl], sem))
                cps.append(pltpu.async_copy(x_hbm.at[sia.at[row]],
                                            bv.at[sl], sem))
            return cps

        pend = [fire(0), None]
        for b in range(NB2):
            if b + 1 < NB2:
                pend[(b + 1) % 2] = fire(b + 1)
            for cp in pend[b % 2]:
                cp.wait()
            bu, bv, _ = bufs[b % 2]
            off = w * EPW + b * GB2
            pltpu.sync_copy(bu, od_hbm.at[pl.ds(off, GB2)])
            pltpu.sync_copy(bv, os_hbm.at[pl.ds(off, GB2)])

    return k(X, idx_d, idx_s)


def _sc_gather_rows(T, idx, H):
    @functools.partial(
        pl.kernel,
        mesh=_mesh,
        compiler_params=_sc_params,
        out_type=jax.ShapeDtypeStruct((EPAD, H), jnp.float32),
        scratch_types=[
            pltpu.VMEM((RPW, IB), jnp.int32),
            pltpu.VMEM((GB2, H), jnp.float32),
            pltpu.VMEM((GB2, H), jnp.float32),
            pltpu.SemaphoreType.DMA,
            pltpu.SemaphoreType.DMA,
        ],
    )
    def k(t_hbm, i_hbm, o_hbm, dia, bu0, bu1, sem0, sem1):
        w = lax.axis_index("s") * 2 + lax.axis_index("c")
        pltpu.sync_copy(i_hbm.at[pl.ds(w * RPW, RPW)], dia)
        bufs = ((bu0, sem0), (bu1, sem1))

        def fire(b):
            bu, sem = bufs[b % 2]
            cps = []
            for j in range(GB2 // IB):
                row = b * (GB2 // IB) + j
                sl = pl.ds(j * IB, IB)
                cps.append(pltpu.async_copy(t_hbm.at[dia.at[row]],
                                            bu.at[sl], sem))
            return cps

        pend = [fire(0), None]
        for b in range(NB2):
            if b + 1 < NB2:
                pend[(b + 1) % 2] = fire(b + 1)
            for cp in pend[b % 2]:
                cp.wait()
            bu, _ = bufs[b % 2]
            pltpu.sync_copy(bu, o_hbm.at[pl.ds(w * EPW + b * GB2, GB2)])

    return k(T, idx)


def _sc_segmax(M, dstp, bounds, H):
    @functools.partial(
        pl.kernel,
        mesh=_mesh,
        compiler_params=_sc_params,
        out_type=jax.ShapeDtypeStruct((NPAD, H), jnp.float32),
        scratch_types=[
            pltpu.VMEM((NW, 16), jnp.int32),
            pltpu.VMEM((GB,), jnp.int32),
            pltpu.VMEM((GB, H), jnp.float32),
            pltpu.VMEM((NPW + 1, H), jnp.float32),
        ],
    )
    def k(m_hbm, d_hbm, b_hbm, o_hbm, bnd, dsm, mbuf, acc):
        w = lax.axis_index("s") * 2 + lax.axis_index("c")
        pltpu.sync_copy(b_hbm, bnd)

        @pl.loop(0, NPW + 1)
        def _(r):
            for c in range(H // 16):
                acc[r, pl.ds(c * 16, 16)] = jnp.zeros((16,), jnp.float32)

        bv = bnd[w, pl.ds(0, 16)]
        start = bv[0]
        end = bv[1]
        nbase = w * NPW

        abase = (start // 8) * 8

        @pl.loop(0, EPAD // GB)
        def _(i):
            base = pl.multiple_of(abase + i * GB, 8)

            @pl.when(base < end)
            def _():
                pltpu.sync_copy(m_hbm.at[pl.ds(base, GB)], mbuf)
                pltpu.sync_copy(d_hbm.at[pl.ds(base, GB)], dsm)

                @pl.loop(0, GB // 16)
                def _(g):
                    dvec = dsm[pl.ds(g * 16, 16)] - nbase
                    for lane in range(16):
                        kk = base + g * 16 + lane
                        ok = jnp.logical_and(kk >= start, kk < end)
                        loc = jnp.where(ok, dvec[lane], NPW)
                        r = g * 16 + lane
                        for c in range(H // 16):
                            cs = pl.ds(c * 16, 16)
                            acc[loc, cs] = jnp.maximum(acc[loc, cs],
                                                       mbuf[r, cs])
        pltpu.sync_copy(acc.at[pl.ds(0, NPW)], o_hbm.at[pl.ds(nbase, NPW)])

    return k(M, dstp, bounds)


def _tc_head(coord, feat, jt, p):
    (wh1, bh1, wh2, bh2, a0, a1, a2, v0, v1, v2, b1) = p

    def body(c_r, f_r, j_r, wh1_r, bh1_r, wh2_r, bh2_r, a0_r, a1_r, a2_r,
             v0_r, v1_r, v2_r, b1_r, u_o, v_o):
        x0 = _relu(jnp.dot(c_r[...], wh1_r[...],
                           preferred_element_type=jnp.float32) + bh1_r[...])
        x0 = _relu(jnp.dot(x0, wh2_r[...],
                           preferred_element_type=jnp.float32) + bh2_r[...])
        du = jnp.dot(x0, a0_r[...], preferred_element_type=jnp.float32)
        du += jnp.dot(f_r[...], a1_r[...], preferred_element_type=jnp.float32)
        du += jnp.dot(j_r[...], a2_r[...], preferred_element_type=jnp.float32)
        dv = jnp.dot(x0, v0_r[...], preferred_element_type=jnp.float32)
        dv += jnp.dot(f_r[...], v1_r[...], preferred_element_type=jnp.float32)
        dv += jnp.dot(j_r[...], v2_r[...], preferred_element_type=jnp.float32)
        u_o[...] = du - dv + b1_r[...]
        v_o[...] = dv

    H = a0.shape[1]
    RB = 1000
    full = lambda a: pl.BlockSpec(a.shape, lambda i: tuple(0 for _ in a.shape))
    return pl.pallas_call(
        body,
        grid=(N // RB,),
        in_specs=[
            pl.BlockSpec((RB, 3), lambda i: (i, 0)),
            pl.BlockSpec((RB, 512), lambda i: (i, 0)),
            pl.BlockSpec((RB, 16), lambda i: (i, 0)),
            full(wh1), full(bh1), full(wh2), full(bh2),
            full(a0), full(a1), full(a2),
            full(v0), full(v1), full(v2), full(b1),
        ],
        out_specs=(pl.BlockSpec((RB, H), lambda i: (i, 0)),
                   pl.BlockSpec((RB, H), lambda i: (i, 0))),
        out_shape=(jax.ShapeDtypeStruct((N, H), jnp.float32),
                   jax.ShapeDtypeStruct((N, H), jnp.float32)),
    )(coord, feat, jt, wh1, bh1, wh2, bh2, a0, a1, a2, v0, v1, v2, b1)


def _tc_edge_proj(ea_s, we1, we2):
    def body(e_r, w1_r, w2_r, c1_o, c2_o):
        c1_o[...] = jnp.dot(e_r[...], w1_r[...],
                            preferred_element_type=jnp.float32)
        c2_o[...] = jnp.dot(e_r[...], w2_r[...],
                            preferred_element_type=jnp.float32)

    H = we1.shape[1]
    g = EPAD // GB
    return pl.pallas_call(
        body,
        grid=(g,),
        in_specs=[
            pl.BlockSpec((GB, 16), lambda i: (i, 0)),
            pl.BlockSpec((16, H), lambda i: (0, 0)),
            pl.BlockSpec((16, H), lambda i: (0, 0)),
        ],
        out_specs=(pl.BlockSpec((GB, H), lambda i: (i, 0)),
                   pl.BlockSpec((GB, H), lambda i: (i, 0))),
        out_shape=(jax.ShapeDtypeStruct((EPAD, H), jnp.float32),
                   jax.ShapeDtypeStruct((EPAD, H), jnp.float32)),
    )(ea_s, we1, we2)


def _tc_mlp2(G0, C, w2, b2):
    H = w2.shape[1]
    g = EPAD // GB

    if C is not None:
        def body(g_r, c_r, w_r, b_r, m_o):
            h = _relu(g_r[...] + c_r[...])
            m_o[...] = _relu(jnp.dot(h, w_r[...],
                                     preferred_element_type=jnp.float32)
                             + b_r[...])
        in_specs = [
            pl.BlockSpec((GB, H), lambda i: (i, 0)),
            pl.BlockSpec((GB, H), lambda i: (i, 0)),
            pl.BlockSpec((H, H), lambda i: (0, 0)),
            pl.BlockSpec((1, H), lambda i: (0, 0)),
        ]
        args = (G0, C, w2, b2)
    else:
        def body(g_r, w_r, b_r, m_o):
            h = _relu(g_r[...])
            m_o[...] = _relu(jnp.dot(h, w_r[...],
                                     preferred_element_type=jnp.float32)
                             + b_r[...])
        in_specs = [
            pl.BlockSpec((GB, H), lambda i: (i, 0)),
            pl.BlockSpec((H, H), lambda i: (0, 0)),
            pl.BlockSpec((1, H), lambda i: (0, 0)),
        ]
        args = (G0, w2, b2)

    return pl.pallas_call(
        body,
        grid=(g,),
        in_specs=in_specs,
        out_specs=pl.BlockSpec((GB, H), lambda i: (i, 0)),
        out_shape=jax.ShapeDtypeStruct((EPAD + GB, H), jnp.float32),
    )(*args)


def _tc_mlp_ref(Gd, Gs, C, wi, wj, b1, w2, b2):
    H = w2.shape[1]
    g = EPAD // GB

    if C is not None:
        def body(gd_r, gs_r, c_r, wi_r, wj_r, b1_r, w2_r, b2_r, m_o):
            xi = gd_r[...]
            dj = gs_r[...] - xi
            h = jnp.dot(xi, wi_r[...], preferred_element_type=jnp.float32)
            h += jnp.dot(dj, wj_r[...], preferred_element_type=jnp.float32)
            h = _relu(h + c_r[...] + b1_r[...])
            m_o[...] = _relu(jnp.dot(h, w2_r[...],
                                     preferred_element_type=jnp.float32)
                             + b2_r[...])
        in_specs = [
            pl.BlockSpec((GB, wi.shape[0]), lambda i: (i, 0)),
            pl.BlockSpec((GB, wi.shape[0]), lambda i: (i, 0)),
            pl.BlockSpec((GB, H), lambda i: (i, 0)),
            pl.BlockSpec(wi.shape, lambda i: (0, 0)),
            pl.BlockSpec(wj.shape, lambda i: (0, 0)),
            pl.BlockSpec((1, H), lambda i: (0, 0)),
            pl.BlockSpec((H, H), lambda i: (0, 0)),
            pl.BlockSpec((1, H), lambda i: (0, 0)),
        ]
        args = (Gd, Gs, C, wi, wj, b1, w2, b2)
    else:
        def body(gd_r, gs_r, wi_r, wj_r, b1_r, w2_r, b2_r, m_o):
            xi = gd_r[...]
            dj = gs_r[...] - xi
            h = jnp.dot(xi, wi_r[...], preferred_element_type=jnp.float32)
            h += jnp.dot(dj, wj_r[...], preferred_element_type=jnp.float32)
            h = _relu(h + b1_r[...])
            m_o[...] = _relu(jnp.dot(h, w2_r[...],
                                     preferred_element_type=jnp.float32)
                             + b2_r[...])
        in_specs = [
            pl.BlockSpec((GB, wi.shape[0]), lambda i: (i, 0)),
            pl.BlockSpec((GB, wi.shape[0]), lambda i: (i, 0)),
            pl.BlockSpec(wi.shape, lambda i: (0, 0)),
            pl.BlockSpec(wj.shape, lambda i: (0, 0)),
            pl.BlockSpec((1, H), lambda i: (0, 0)),
            pl.BlockSpec((H, H), lambda i: (0, 0)),
            pl.BlockSpec((1, H), lambda i: (0, 0)),
        ]
        args = (Gd, Gs, wi, wj, b1, w2, b2)

    return pl.pallas_call(
        body,
        grid=(g,),
        in_specs=in_specs,
        out_specs=pl.BlockSpec((GB, H), lambda i: (i, 0)),
        out_shape=jax.ShapeDtypeStruct((EPAD + GB, H), jnp.float32),
    )(*args)


def _tc_dense_relu(x, w, b):
    def body(x_r, w_r, b_r, o_r):
        o_r[...] = _relu(jnp.dot(x_r[...], w_r[...],
                                 preferred_element_type=jnp.float32) + b_r[...])

    return pl.pallas_call(
        body,
        out_shape=jax.ShapeDtypeStruct((N, w.shape[1]), jnp.float32),
    )(x, w, b)


def _tc_add(a, b):
    def body(a_r, b_r, o_r):
        o_r[...] = a_r[...] + b_r[...]

    return pl.pallas_call(
        body,
        out_shape=jax.ShapeDtypeStruct(a.shape, jnp.float32),
    )(a, b)


def _tc_tail(a5, a4, wo1, bo1, wo2, bo2):
    def body(a5_r, a4_r, w1_r, b1_r, w2_r, b2_r, y1_o, y2_o):
        xo = a5_r[...] + a4_r[...]
        y1 = _relu(jnp.dot(xo, w1_r[...],
                           preferred_element_type=jnp.float32) + b1_r[...])
        y1_o[...] = _relu(jnp.dot(y1, w1_r[...],
                                  preferred_element_type=jnp.float32) + b1_r[...])
        y2 = _relu(jnp.dot(xo, w2_r[...],
                           preferred_element_type=jnp.float32) + b2_r[...])
        y2_o[...] = _relu(jnp.dot(y2, w2_r[...],
                                  preferred_element_type=jnp.float32) + b2_r[...])

    H = wo1.shape[1]
    return pl.pallas_call(
        body,
        out_shape=(jax.ShapeDtypeStruct((N, H), jnp.float32),
                   jax.ShapeDtypeStruct((N, H), jnp.float32)),
    )(a5, a4, wo1, bo1, wo2, bo2)


def kernel(Coordinate3D, Feature512D, JointType, edge_attr, params, edge_index):
    i32 = jnp.int32
    src = edge_index[0].astype(i32)
    dst = edge_index[1].astype(i32)

    perm = jnp.argsort(dst)
    dst_s = dst[perm]
    src_s = src[perm]
    padE = EPAD - E
    zpad = jnp.zeros((padE,), i32)
    gd = jnp.concatenate([dst_s, zpad]).reshape(EPAD // IB, IB)
    gs = jnp.concatenate([src_s, zpad]).reshape(EPAD // IB, IB)
    gp = jnp.concatenate([perm.astype(i32), zpad]).reshape(EPAD // IB, IB)
    dstp = jnp.concatenate(
        [dst_s, jnp.full((padE + GB,), NPAD - 1, i32)])
    bnd1 = jnp.searchsorted(
        dstp[:EPAD], jnp.arange(0, NPW * (NW + 1), NPW)).astype(i32)
    bounds = jnp.zeros((NW, 16), i32)
    bounds = bounds.at[:, 0].set(bnd1[:NW]).at[:, 1].set(bnd1[1:])

    W1c = params["ece1_1"]["W"]
    au1 = W1c[:560]
    av1 = W1c[560:1120]
    we1 = W1c[1120:]
    b1_1 = params["ece1_1"]["b"].reshape(1, -1)
    we2 = params["ece2_1"]["W"][2 * 64:]
    r2 = lambda p: (p["W"], p["b"].reshape(1, -1))

    head_p = (params["h1"]["W"], params["h1"]["b"].reshape(1, -1),
              params["h2"]["W"], params["h2"]["b"].reshape(1, -1),
              au1[:32], au1[32:544], au1[544:],
              av1[:32], av1[32:544], av1[544:], b1_1)
    U1, V1 = _tc_head(Coordinate3D, Feature512D, JointType, head_p)

    ea_s = _sc_gather_rows(edge_attr, gp, 16)
    C1, C2 = _tc_edge_proj(ea_s, we1, we2)

    def conv1(U, V, C, p2, H):
        G0 = _sc_gather_sum(U, V, gd, gs, H)
        w2, b2 = r2(p2)
        M = _tc_mlp2(G0, C, w2, b2)
        return _sc_segmax(M, dstp, bounds, H)[:N]

    def conv_ref(x, p1, p2, C, cdim, H):
        W = p1["W"]
        wi = W[:cdim]
        wj = W[cdim:2 * cdim]
        b1 = p1["b"].reshape(1, -1)
        Gd, Gs = _sc_gather_pair(x, gd, gs, cdim)
        w2, b2 = r2(p2)
        M = _tc_mlp_ref(Gd, Gs, C, wi, wj, b1, w2, b2)
        return _sc_segmax(M, dstp, bounds, H)[:N]

    a1 = conv1(U1, V1, C1, params["ece1_2"], 64)
    a2 = conv_ref(a1, params["ece2_1"], params["ece2_2"], C2, 64, 64)
    x3 = _tc_dense_relu(a2, params["h3"]["W"],
                        params["h3"]["b"].reshape(1, -1))
    a3 = conv_ref(x3, params["ec1_1"], params["ec1_2"], None, 48, 48)
    a4 = conv_ref(a3, params["ec2_1"], params["ec2_2"], None, 48, 48)
    x5 = _tc_add(a4, a3)
    a5 = conv_ref(x5, params["ec3_1"], params["ec3_2"], None, 48, 48)

    wo1, bo1 = r2(params["out1"])
    wo2, bo2 = r2(params["out2"])
    return _tc_tail(a5, a4, wo1, bo1, wo2, bo2)
